# baseline (speedup 1.0000x reference)
import functools

import jax
import jax.numpy as jnp
from jax import lax
from jax.experimental import pallas as pl
from jax.experimental.pallas import tpu as pltpu
from jax.experimental.pallas import tpu_sc as plsc

_B, _N, _M, _C = 16, 16384, 64, 1024
_L = 16
_CHUNKS = _N // _L
_BN = 2048
_INT_MAX = 2147483647


def _sc_body(wu_hbm, parts_hbm, row_v, onehot_v):
    c = lax.axis_index("c")
    s = lax.axis_index("s")
    lane = lax.iota(jnp.int32, _L)

    @pl.when(c == 0)
    def _():
        pltpu.sync_copy(wu_hbm.at[s], row_v)
        _U = 4

        def step(i, carry):
            out = []
            for u in range(_U):
                mn, mi = carry[u]
                base = (i * _U + u) * _L
                v = row_v[pl.ds(base, _L)]
                lt = v < mn
                out.append((jnp.where(lt, v, mn),
                            jnp.where(lt, lane + base, mi)))
            return tuple(out)

        init = tuple(
            (jnp.full((_L,), jnp.inf, jnp.float32),
             jnp.zeros((_L,), jnp.int32))
            for _ in range(_U)
        )
        scans = lax.fori_loop(0, _CHUNKS // _U, step, init)
        mn, mi = scans[0]
        for u in range(1, _U):
            vn, vi = scans[u]
            take = (vn < mn) | ((vn == mn) & (vi < mi))
            mn = jnp.where(take, vn, mn)
            mi = jnp.where(take, vi, mi)
        m = jnp.min(mn)
        cand = jnp.where(mn == m, mi, _INT_MAX)
        idx = jnp.min(cand)

        def onehot_step(i, _):
            for u in range(_U):
                base = (i * _U + u) * _L
                onehot_v[pl.ds(base, _L)] = jnp.where(
                    lane + base == idx, 1.0, 0.0)
            return 0

        lax.fori_loop(0, _CHUNKS // _U, onehot_step, 0)
        pltpu.sync_copy(onehot_v, parts_hbm.at[s])


@functools.cache
def _sc_argmin_onehot():
    return pl.kernel(
        _sc_body,
        out_type=jax.ShapeDtypeStruct((_B, _N), jnp.float32),
        compiler_params=pltpu.CompilerParams(needs_layout_passes=False),
        mesh=plsc.VectorSubcoreMesh(
            core_axis_name="c", subcore_axis_name="s",
            num_cores=2, num_subcores=16,
        ),
        scratch_types=[
            pltpu.VMEM((_N,), jnp.float32),
            pltpu.VMEM((_N,), jnp.float32),
        ],
    )


def _tc_body(emb_ref, wfc_ref, bfc_ref, wlu_ref, wr_ref,
             mem_ref, w_out_ref, mem_out_ref, a_ref, kt_ref):
    j = pl.program_id(0)

    @pl.when(j == 0)
    def _():
        o = lax.dot_general(
            emb_ref[...], wfc_ref[...], (((1,), (1,)), ((), ())),
            preferred_element_type=jnp.float32,
        ) + bfc_ref[...]
        beta = o[:, _M:_M + 1]
        a_ref[...] = 1.0 / (1.0 + jnp.exp(-beta))
        kt_ref[...] = o[:, :_M].reshape(_B, _M, 1)

    a = a_ref[...]
    wl = jnp.max(wlu_ref[...], axis=0, keepdims=True)
    wblk = a * wr_ref[...] + (1.0 - a) * wl
    w_out_ref[...] = wblk
    w3 = lax.broadcast_in_dim(wblk, (_B, _M, _BN), (0, 2))
    k3 = lax.broadcast_in_dim(kt_ref[...], (_B, _M, _BN), (0, 1, 2))
    mem_out_ref[...] = mem_ref[...] + w3 * k3


def _tc_dense(emb, wfc, bfc, wlu_parts, w_r_prev, memT):
    return pl.pallas_call(
        _tc_body,
        grid=(_N // _BN,),
        in_specs=[
            pl.BlockSpec((_B, _C), lambda j: (0, 0)),
            pl.BlockSpec((_M + 1, _C), lambda j: (0, 0)),
            pl.BlockSpec((1, _M + 1), lambda j: (0, 0)),
            pl.BlockSpec((_B, _BN), lambda j: (0, j)),
            pl.BlockSpec((_B, _BN), lambda j: (0, j)),
            pl.BlockSpec((_B, _M, _BN), lambda j: (0, 0, j)),
        ],
        out_specs=[
            pl.BlockSpec((_B, _BN), lambda j: (0, j)),
            pl.BlockSpec((_B, _M, _BN), lambda j: (0, 0, j)),
        ],
        out_shape=[
            jax.ShapeDtypeStruct((_B, _N), jnp.float32),
            jax.ShapeDtypeStruct((_B, _M, _N), jnp.float32),
        ],
        scratch_shapes=[
            pltpu.VMEM((_B, 1), jnp.float32),
            pltpu.VMEM((_B, _M, 1), jnp.float32),
        ],
    )(emb, wfc, bfc, wlu_parts, w_r_prev, memT)


def kernel(embeddings, w_r_prev, w_u, memory, W_fc, b_fc):
    wlu_parts = _sc_argmin_onehot()(w_u[0])
    memT = jnp.swapaxes(memory, 1, 2)
    w, memT_new = _tc_dense(embeddings, W_fc, b_fc.reshape(1, _M + 1),
                            wlu_parts, w_r_prev, memT)
    return w, jnp.swapaxes(memT_new, 1, 2)
